# baseline (speedup 1.0000x reference)
import jax
import jax.numpy as jnp
from jax.experimental import pallas as pl
from jax.experimental.pallas import tpu as pltpu
from jax.experimental.pallas import tpu_sc as plsc

_LANES = 16


def kernel(x, emb_table):
    B, S, D = x.shape
    pos = emb_table[:S]
    S_BLK = 4
    n_sub = 32
    grid = (n_sub, S // S_BLK // n_sub)
    buf_x = pl.Buffered(buffer_count=5)
    buf_e = pl.Buffered(buffer_count=3)

    vector_mesh = plsc.VectorSubcoreMesh(
        core_axis_name="core", subcore_axis_name="subcore"
    )

    @pl.kernel(out_type=jax.ShapeDtypeStruct((B, S, D), x.dtype),
               mesh=vector_mesh)
    def sc_add(x_hbm, emb_hbm, o_hbm):
        def body(x_vmem, emb_vmem, o_vmem):
            @pl.loop(0, S_BLK)
            def _(r):
                @plsc.parallel_loop(0, D, step=_LANES, unroll=8)
                def _(c):
                    e = emb_vmem.at[r, pl.ds(c, _LANES)][...]
                    for b in range(B):
                        o_vmem.at[b, r, pl.ds(c, _LANES)][...] = (
                            x_vmem.at[b, r, pl.ds(c, _LANES)][...] + e
                        )

        pltpu.emit_pipeline(
            body,
            grid=grid,
            in_specs=[
                pl.BlockSpec((B, S_BLK, D), lambda p, i: (0, i * 32 + p, 0),
                             pipeline_mode=buf_x),
                pl.BlockSpec((S_BLK, D), lambda p, i: (i * 32 + p, 0),
                             pipeline_mode=buf_e),
            ],
            out_specs=[pl.BlockSpec((B, S_BLK, D), lambda p, i: (0, i * 32 + p, 0))],
            core_axis_name=("core", "subcore"),
            dimension_semantics=(pltpu.PARALLEL, pltpu.ARBITRARY),
        )(x_hbm, emb_hbm, o_hbm)

    return sc_add(x, pos)


# speedup vs baseline: 1.0138x; 1.0138x over previous
import jax
import jax.numpy as jnp
from jax.experimental import pallas as pl
from jax.experimental.pallas import tpu as pltpu
from jax.experimental.pallas import tpu_sc as plsc

_LANES = 16


def kernel(x, emb_table):
    B, S, D = x.shape
    pos = emb_table[:S]
    S_BLK = 4
    grid = (S // S_BLK,)
    buf_x = pl.Buffered(buffer_count=5)
    buf_e = pl.Buffered(buffer_count=3)

    vector_mesh = plsc.VectorSubcoreMesh(
        core_axis_name="core", subcore_axis_name="subcore"
    )

    @pl.kernel(out_type=jax.ShapeDtypeStruct((B, S, D), x.dtype),
               mesh=vector_mesh)
    def sc_add(x_hbm, emb_hbm, o_hbm):
        def body(x_vmem, emb_vmem, o_vmem):
            @pl.loop(0, S_BLK)
            def _(r):
                @plsc.parallel_loop(0, D, step=_LANES, unroll=8)
                def _(c):
                    e = emb_vmem.at[r, pl.ds(c, _LANES)][...]
                    for b in range(B):
                        o_vmem.at[b, r, pl.ds(c, _LANES)][...] = (
                            x_vmem.at[b, r, pl.ds(c, _LANES)][...] + e
                        )

        pltpu.emit_pipeline(
            body,
            grid=grid,
            in_specs=[
                pl.BlockSpec((B, S_BLK, D), lambda i: (0, i, 0),
                             pipeline_mode=buf_x),
                pl.BlockSpec((S_BLK, D), lambda i: (i, 0),
                             pipeline_mode=buf_e),
            ],
            out_specs=[pl.BlockSpec((B, S_BLK, D), lambda i: (0, i, 0))],
            core_axis_name=("core", "subcore"),
            dimension_semantics=(pltpu.PARALLEL,),
        )(x_hbm, emb_hbm, o_hbm)

    return sc_add(x, pos)
